# Initial kernel scaffold; baseline (speedup 1.0000x reference)
import jax
import jax.numpy as jnp
from jax.experimental import pallas as pl


def kernel(value_samples, weights_samples, cu_seqlens):
    raise NotImplementedError("write your pallas kernel here")


# speedup vs baseline: 682.0973x; 682.0973x over previous
import functools

import jax
import jax.numpy as jnp
from jax import lax
from jax.experimental import pallas as pl
from jax.experimental.pallas import tpu as pltpu
from jax.experimental.pallas import tpu_sc as plsc

LANES = 128
BR = 256
NW = 32


def _cumsum_body(v_ref, w_ref, o0_ref, o1_ref, o2_ref, carry_ref):
    b = pl.program_id(0)

    @pl.when(b == 0)
    def _init():
        carry_ref[0] = 0.0
        carry_ref[1] = 0.0
        carry_ref[2] = 0.0

    ii = lax.broadcasted_iota(jnp.int32, (LANES, LANES), 0)
    jj = lax.broadcasted_iota(jnp.int32, (LANES, LANES), 1)
    upper_incl = (ii <= jj).astype(jnp.float32)
    aa = lax.broadcasted_iota(jnp.int32, (BR, BR), 0)
    bb = lax.broadcasted_iota(jnp.int32, (BR, BR), 1)
    strict_lower = (bb < aa).astype(jnp.float32)

    w = w_ref[...]
    outs = (o0_ref, o1_ref, o2_ref)
    for c in range(3):
        x = v_ref[c] * w
        incl = jnp.dot(x, upper_incl, preferred_element_type=jnp.float32,
                       precision=lax.Precision.HIGHEST)
        row_sums = incl[:, LANES - 1:LANES]
        row_carry = jnp.dot(strict_lower, row_sums,
                            preferred_element_type=jnp.float32,
                            precision=lax.Precision.HIGHEST)
        outs[c][...] = incl + row_carry + carry_ref[c]
        carry_ref[c] = carry_ref[c] + jnp.sum(row_sums)


def _stage_a(v3, w2):
    rows = v3.shape[1]
    nblocks = rows // BR
    return pl.pallas_call(
        _cumsum_body,
        grid=(nblocks,),
        in_specs=[
            pl.BlockSpec((3, BR, LANES), lambda b: (0, b, 0)),
            pl.BlockSpec((BR, LANES), lambda b: (b, 0)),
        ],
        out_specs=[pl.BlockSpec((BR, LANES), lambda b: (b, 0))] * 3,
        out_shape=[jax.ShapeDtypeStruct((rows, LANES), jnp.float32)] * 3,
        scratch_shapes=[pltpu.SMEM((3,), jnp.float32)],
    )(v3, w2)


def _make_stage_b(n_rays, chunk):
    rpw = n_rays // NW
    mesh = plsc.VectorSubcoreMesh(core_axis_name="c", subcore_axis_name="s")

    @functools.partial(
        pl.kernel,
        mesh=mesh,
        out_type=[jax.ShapeDtypeStruct((n_rays,), jnp.float32)] * 3,
        scratch_types=[
            pltpu.VMEM((chunk,), jnp.int32),
            pltpu.VMEM((chunk,), jnp.int32),
            pltpu.VMEM((chunk,), jnp.float32),
            pltpu.VMEM((chunk,), jnp.float32),
            pltpu.VMEM((rpw,), jnp.float32),
            pltpu.SemaphoreType.DMA,
        ],
    )
    def stage_b(c0h, c1h, c2h, cuh, o0h, o1h, o2h,
                cu_v, idx_v, m_v, g_v, out_v, sem):
        wid = lax.axis_index("s") * 2 + lax.axis_index("c")
        base = wid * rpw
        pltpu.sync_copy(cuh.at[pl.ds(base, chunk)], cu_v)
        for q in range(0, chunk, 16):
            cu16 = cu_v[pl.ds(q, 16)]
            idx_v[pl.ds(q, 16)] = jnp.maximum(cu16 - 1, 0)
            m_v[pl.ds(q, 16)] = jnp.where(cu16 > 0, 1.0, 0.0)
        for ch_h, oh in ((c0h, o0h), (c1h, o1h), (c2h, o2h)):
            pltpu.async_copy(ch_h.at[idx_v], g_v, sem).wait()
            for q in range(0, rpw, 16):
                glo = g_v[pl.ds(q, 16)] * m_v[pl.ds(q, 16)]
                ghi = g_v[pl.ds(q + 1, 16)] * m_v[pl.ds(q + 1, 16)]
                out_v[pl.ds(q, 16)] = ghi - glo
            pltpu.sync_copy(out_v, oh.at[pl.ds(base, rpw)])

    return stage_b


def kernel(value_samples, weights_samples, cu_seqlens):
    total = value_samples.shape[0]
    n_rays = cu_seqlens.shape[0] - 1
    rows = total // LANES

    v3 = value_samples.T.reshape(3, rows, LANES)
    w2 = weights_samples.reshape(rows, LANES)
    c0, c1, c2 = _stage_a(v3, w2)

    rpw = n_rays // NW
    chunk = rpw + 16
    pad = NW * rpw + chunk - (n_rays + 1)
    cu_pad = jnp.concatenate(
        [cu_seqlens.astype(jnp.int32),
         jnp.full((pad,), total, dtype=jnp.int32)])

    stage_b = _make_stage_b(n_rays, chunk)
    o0, o1, o2 = stage_b(c0.reshape(total), c1.reshape(total),
                         c2.reshape(total), cu_pad)
    return jnp.stack([o0, o1, o2], axis=1)
